# baseline (speedup 1.0000x reference)
import functools

import jax
import jax.numpy as jnp
from jax import lax
from jax.experimental import pallas as pl
from jax.experimental.pallas import tpu as pltpu
from jax.experimental.pallas import tpu_sc as plsc

_NC = 2
_NS = 16
_NW = _NC * _NS
_LANES = 16


def _sc_fused(table_flat, ids_flat, strand_ids, strand_table, pos_table,
              ln_gamma, ln_beta, Hn, V, D):
    CW = 128
    BH = ids_flat.shape[0]
    per_w = BH // CW // _NW
    b_per_w = BH // _NW // Hn
    C16 = D // _LANES

    mesh = plsc.VectorSubcoreMesh(
        core_axis_name="c", subcore_axis_name="s",
        num_cores=_NC, num_subcores=_NS,
    )

    @functools.partial(
        pl.kernel,
        out_type=jax.ShapeDtypeStruct((BH, D), jnp.float32),
        mesh=mesh,
        scratch_types=[
            pltpu.VMEM((CW,), jnp.int32),
            pltpu.VMEM((CW,), jnp.int32),
            pltpu.VMEM((CW, D), jnp.float32),
            pltpu.VMEM((CW, D), jnp.float32),
            pltpu.VMEM((CW, D), jnp.float32),
            pltpu.VMEM((CW, D), jnp.float32),
            pltpu.VMEM((per_w * CW,), jnp.int32),
            pltpu.VMEM((2 * Hn, D), jnp.float32),
            pltpu.VMEM((D,), jnp.float32),
            pltpu.VMEM((D,), jnp.float32),
            pltpu.VMEM((2, D), jnp.float32),
            pltpu.VMEM((Hn, D), jnp.float32),
            pltpu.SemaphoreType.DMA,
            pltpu.SemaphoreType.DMA,
            pltpu.SemaphoreType.DMA,
            pltpu.SemaphoreType.DMA,
        ],
    )
    def k(tab_hbm, ids_hbm, aidx_hbm, st_hbm, pos_hbm, gam_hbm, bet_hbm,
          out_hbm, idx_a, idx_b, rows_a, rows_b, res_a, res_b,
          aidx_v, posS_v, gam_v, bet_v, st_v, pos_v,
          gsem0, gsem1, wsem0, wsem1):
        wid = lax.axis_index("s") * _NC + lax.axis_index("c")
        base = pl.multiple_of(wid * (per_w * CW), per_w * CW)

        pltpu.sync_copy(aidx_hbm.at[pl.ds(base, per_w * CW)], aidx_v)
        pltpu.sync_copy(st_hbm, st_v)
        pltpu.sync_copy(pos_hbm, pos_v)
        pltpu.sync_copy(gam_hbm, gam_v)
        pltpu.sync_copy(bet_hbm, bet_v)

        def build_posS(h, carry):
            for c in range(C16):
                sl = pl.ds(c * _LANES, _LANES)
                p = pos_v[h, sl]
                posS_v[h, sl] = p + st_v[0, sl]
                posS_v[Hn + h, sl] = p + st_v[1, sl]
            return carry

        lax.fori_loop(0, Hn, build_posS, 0)

        gam = [gam_v[pl.ds(c * _LANES, _LANES)] for c in range(C16)]
        bet = [bet_v[pl.ds(c * _LANES, _LANES)] for c in range(C16)]

        magic = jnp.full((16,), 0x5f3759df, dtype=jnp.int32)
        one_v = jnp.full((16,), 1, dtype=jnp.int32)
        inv_d = jnp.float32(1.0 / D)
        half = jnp.float32(0.5)
        onep5 = jnp.float32(1.5)
        eps = jnp.float32(1e-5)
        perms = [jnp.arange(16, dtype=jnp.int32) ^ (1 << j) for j in range(4)]

        def allsum(v):
            for pm in perms:
                v = v + v.at[pm].get(mode="promise_in_bounds")
            return v

        def prep_idx(i, idx_s):
            def pc(c, carry2):
                sl = pl.ds(pl.multiple_of(c * _LANES, _LANES), _LANES)
                p = lax.iota(jnp.int32, 16) + (base + i * CW + c * _LANES)
                h = lax.rem(p, Hn)
                idx_s[sl] = idx_s[sl] + h * V
                return carry2

            lax.fori_loop(0, CW // _LANES, pc, 0, unroll=True)

        def compute_chunk(i, rows_s, res_s):
            def _grp(g, carry2):
                gof = pl.multiple_of(i * CW + g * _LANES, _LANES)
                av = aidx_v[pl.ds(gof, _LANES)]
                for r16 in range(_LANES):
                    r = g * _LANES + r16
                    a = av[r16]
                    x = []
                    for c in range(C16):
                        sl = pl.ds(c * _LANES, _LANES)
                        x.append(rows_s[r, sl] + posS_v[a, sl])
                    sv = x[0]
                    for c in range(1, C16):
                        sv = sv + x[c]
                    qv = x[0] * x[0]
                    for c in range(1, C16):
                        qv = qv + x[c] * x[c]
                    mean = allsum(sv) * inv_d
                    ex2 = allsum(qv) * inv_d
                    var = ex2 - mean * mean + eps
                    yi = magic - lax.shift_right_arithmetic(
                        lax.bitcast_convert_type(var, jnp.int32), one_v)
                    y = lax.bitcast_convert_type(yi, jnp.float32)
                    hv = half * var
                    y = y * (onep5 - hv * y * y)
                    y = y * (onep5 - hv * y * y)
                    for c in range(C16):
                        sl = pl.ds(c * _LANES, _LANES)
                        res_s[r, sl] = (x[c] - mean) * y * gam[c] + bet[c]
                return carry2

            lax.fori_loop(0, CW // _LANES, _grp, 0)

        def body(i, carry):
            off = pl.multiple_of(base + i * CW, CW)
            pltpu.sync_copy(ids_hbm.at[pl.ds(off, CW)], idx_a)
            prep_idx(i, idx_a)
            pltpu.async_copy(tab_hbm.at[idx_a], rows_a, gsem0)
            pltpu.make_async_copy(tab_hbm.at[idx_a], rows_a, gsem0).wait()
            compute_chunk(i, rows_a, res_a)
            pltpu.sync_copy(res_a, out_hbm.at[pl.ds(off, CW)])
            return carry

        lax.fori_loop(0, per_w, body, 0)

    aidx_flat = (jnp.repeat(strand_ids, Hn) * Hn
                 + jnp.tile(jnp.arange(Hn, dtype=jnp.int32), BH // Hn))
    return k(table_flat, ids_flat, aidx_flat, strand_table, pos_table,
             ln_gamma, ln_beta)


def kernel(cluster_ids, strand_ids, cluster_tables, strand_table, pos_table,
           ln_gamma, ln_beta):
    B, Hn = cluster_ids.shape
    _, V, D = cluster_tables.shape
    table_flat = cluster_tables.reshape(Hn * V, D)
    ids_flat = cluster_ids.reshape(B * Hn)
    out2 = _sc_fused(table_flat, ids_flat, strand_ids, strand_table, pos_table,
                     ln_gamma, ln_beta, Hn, V, D)
    return out2.reshape(B, Hn, D)


# speedup vs baseline: 1.7486x; 1.7486x over previous
import functools

import jax
import jax.numpy as jnp
from jax import lax
from jax.experimental import pallas as pl
from jax.experimental.pallas import tpu as pltpu
from jax.experimental.pallas import tpu_sc as plsc

_NC = 2
_NS = 16
_NW = _NC * _NS
_LANES = 16
_NCHUNK = 4
_BB = 32


def _sc_gather(table_flat, ids_flat, base_row, n_rows, Hn, V, D):
    per_w = n_rows // _NW
    CH = 128
    n_iter = per_w // CH

    mesh = plsc.VectorSubcoreMesh(
        core_axis_name="c", subcore_axis_name="s",
        num_cores=_NC, num_subcores=_NS,
    )

    @functools.partial(
        pl.kernel,
        out_type=jax.ShapeDtypeStruct((n_rows, D), jnp.float32),
        mesh=mesh,
        scratch_types=[
            pltpu.VMEM((CH,), jnp.int32),
            pltpu.VMEM((CH, D), jnp.float32),
            pltpu.SemaphoreType.DMA,
        ],
    )
    def k(ids_hbm, table_hbm, out_hbm, idx_v, rows_v, sem):
        wid = lax.axis_index("s") * _NC + lax.axis_index("c")
        base = wid * per_w

        def body(i, carry):
            start = base + i * CH
            pltpu.sync_copy(ids_hbm.at[pl.ds(base_row + start, CH)], idx_v)

            def off_body(j, carry2):
                p = lax.iota(jnp.int32, 16) + (base_row + start + j * _LANES)
                h = lax.rem(p, Hn)
                sl = pl.ds(j * _LANES, _LANES)
                idx_v[sl] = idx_v[sl] + h * V
                return carry2

            lax.fori_loop(0, CH // _LANES, off_body, 0, unroll=True)
            pltpu.async_copy(table_hbm.at[idx_v], rows_v, sem).wait()
            pltpu.sync_copy(rows_v, out_hbm.at[pl.ds(start, CH)])
            return carry

        lax.fori_loop(0, n_iter, body, 0)

    return k(ids_flat, table_flat)


def _tc_post(gathered, strand_ids3, strand_table, pos_table, ln_gamma,
             ln_beta, out_prev, chunk, nchunk):
    Bc, Hn, D = gathered.shape
    B = Bc * nchunk
    grid = (Bc // _BB,)
    goff = chunk * (Bc // _BB)

    have_prev = out_prev is not None

    def body(g_ref, s_ref, st_ref, pt_ref, gm_ref, bt_ref, *rest):
        o_ref = rest[-1]
        x = g_ref[...]
        s = s_ref[0, 0, :].astype(jnp.float32)
        st = st_ref[...]
        semb = st[0][None, :] + s[:, None] * (st[1] - st[0])[None, :]
        x = x + pt_ref[...][None, :, :] + semb[:, None, :]
        mean = jnp.mean(x, axis=-1, keepdims=True)
        xc = x - mean
        var = jnp.mean(xc * xc, axis=-1, keepdims=True)
        y = xc * lax.rsqrt(var + 1e-5)
        o_ref[...] = y * gm_ref[...][None, None, :] + bt_ref[...][None, None, :]

    in_specs = [
        pl.BlockSpec((_BB, Hn, D), lambda i: (i, 0, 0)),
        pl.BlockSpec((1, 1, _BB), lambda i: (i, 0, 0)),
        pl.BlockSpec((2, D), lambda i: (0, 0)),
        pl.BlockSpec((Hn, D), lambda i: (0, 0)),
        pl.BlockSpec((D,), lambda i: (0,)),
        pl.BlockSpec((D,), lambda i: (0,)),
    ]
    args = [gathered, strand_ids3, strand_table, pos_table, ln_gamma, ln_beta]
    if have_prev:
        in_specs.append(pl.BlockSpec(memory_space=pl.ANY))
        args.append(out_prev)
    return pl.pallas_call(
        body,
        grid=grid,
        in_specs=in_specs,
        out_specs=pl.BlockSpec((_BB, Hn, D), lambda i: (i + goff, 0, 0)),
        out_shape=jax.ShapeDtypeStruct((B, Hn, D), jnp.float32),
        input_output_aliases={6: 0} if have_prev else {},
    )(*args)


def kernel(cluster_ids, strand_ids, cluster_tables, strand_table, pos_table,
           ln_gamma, ln_beta):
    B, Hn = cluster_ids.shape
    _, V, D = cluster_tables.shape
    table_flat = cluster_tables.reshape(Hn * V, D)
    ids_flat = cluster_ids.reshape(B * Hn)
    Bc = B // _NCHUNK
    rows_c = Bc * Hn
    out = None
    for c in range(_NCHUNK):
        g = _sc_gather(table_flat, ids_flat, c * rows_c, rows_c, Hn, V, D)
        s3 = lax.dynamic_slice_in_dim(strand_ids, c * Bc, Bc).reshape(
            Bc // _BB, 1, _BB)
        out = _tc_post(g.reshape(Bc, Hn, D), s3, strand_table, pos_table,
                       ln_gamma, ln_beta, out, c, _NCHUNK)
    return out


# speedup vs baseline: 2.2067x; 1.2620x over previous
import functools

import jax
import jax.numpy as jnp
from jax import lax
from jax.experimental import pallas as pl
from jax.experimental.pallas import tpu as pltpu
from jax.experimental.pallas import tpu_sc as plsc

_NC = 2
_NS = 16
_NW = _NC * _NS
_LANES = 16
_NCHUNK = 4
_BB = 32


def _sc_gather(table_flat, ids_flat, base_row, n_rows, Hn, V, D):
    per_w = n_rows // _NW
    CH = 128
    n_iter = per_w // CH

    mesh = plsc.VectorSubcoreMesh(
        core_axis_name="c", subcore_axis_name="s",
        num_cores=_NC, num_subcores=_NS,
    )

    @functools.partial(
        pl.kernel,
        out_type=jax.ShapeDtypeStruct((n_rows, D), jnp.float32),
        mesh=mesh,
        scratch_types=[
            pltpu.VMEM((CH,), jnp.int32),
            pltpu.VMEM((CH, D), jnp.float32),
            pltpu.SemaphoreType.DMA,
        ],
    )
    def k(ids_hbm, table_hbm, out_hbm, idx_v, rows_v, sem):
        wid = lax.axis_index("s") * _NC + lax.axis_index("c")
        base = wid * per_w

        def body(i, carry):
            start = base + i * CH
            pltpu.sync_copy(ids_hbm.at[pl.ds(base_row + start, CH)], idx_v)

            def off_body(j, carry2):
                p = lax.iota(jnp.int32, 16) + (base_row + start + j * _LANES)
                h = lax.rem(p, Hn)
                sl = pl.ds(j * _LANES, _LANES)
                idx_v[sl] = idx_v[sl] + h * V
                return carry2

            lax.fori_loop(0, CH // _LANES, off_body, 0, unroll=True)
            pltpu.async_copy(table_hbm.at[idx_v], rows_v, sem).wait()
            pltpu.sync_copy(rows_v, out_hbm.at[pl.ds(start, CH)])
            return carry

        lax.fori_loop(0, n_iter, body, 0)

    return k(ids_flat, table_flat)


def _tc_post(gathered, strand_ids3, strand_table, pos_table, ln_gamma,
             ln_beta, out_prev, chunk, nchunk):
    Bc, Hn, D = gathered.shape
    B = Bc * nchunk
    grid = (Bc // _BB,)
    goff = chunk * (Bc // _BB)

    have_prev = out_prev is not None

    def body(g_ref, s_ref, st_ref, pt_ref, gm_ref, bt_ref, *rest):
        o_ref = rest[-1]
        x = g_ref[...].reshape(_BB, Hn, D)
        s = s_ref[0, 0, :].astype(jnp.float32)
        st = st_ref[...]
        semb = st[0][None, :] + s[:, None] * (st[1] - st[0])[None, :]
        x = x + pt_ref[...][None, :, :] + semb[:, None, :]
        mean = jnp.mean(x, axis=-1, keepdims=True)
        xc = x - mean
        var = jnp.mean(xc * xc, axis=-1, keepdims=True)
        y = xc * lax.rsqrt(var + 1e-5)
        o_ref[...] = y * gm_ref[...][None, None, :] + bt_ref[...][None, None, :]

    in_specs = [
        pl.BlockSpec((_BB * Hn, D), lambda i: (i, 0)),
        pl.BlockSpec((1, 1, _BB), lambda i: (i, 0, 0)),
        pl.BlockSpec((2, D), lambda i: (0, 0)),
        pl.BlockSpec((Hn, D), lambda i: (0, 0)),
        pl.BlockSpec((D,), lambda i: (0,)),
        pl.BlockSpec((D,), lambda i: (0,)),
    ]
    args = [gathered.reshape(Bc * Hn, D), strand_ids3, strand_table, pos_table,
            ln_gamma, ln_beta]
    if have_prev:
        in_specs.append(pl.BlockSpec(memory_space=pl.ANY))
        args.append(out_prev)
    return pl.pallas_call(
        body,
        grid=grid,
        in_specs=in_specs,
        out_specs=pl.BlockSpec((_BB, Hn, D), lambda i: (i + goff, 0, 0)),
        out_shape=jax.ShapeDtypeStruct((B, Hn, D), jnp.float32),
        input_output_aliases={6: 0} if have_prev else {},
    )(*args)


def kernel(cluster_ids, strand_ids, cluster_tables, strand_table, pos_table,
           ln_gamma, ln_beta):
    B, Hn = cluster_ids.shape
    _, V, D = cluster_tables.shape
    table_flat = cluster_tables.reshape(Hn * V, D)
    ids_flat = cluster_ids.reshape(B * Hn)
    Bc = B // _NCHUNK
    rows_c = Bc * Hn
    out = None
    for c in range(_NCHUNK):
        g = _sc_gather(table_flat, ids_flat, c * rows_c, rows_c, Hn, V, D)
        s3 = lax.dynamic_slice_in_dim(strand_ids, c * Bc, Bc).reshape(
            Bc // _BB, 1, _BB)
        out = _tc_post(g.reshape(Bc, Hn, D), s3, strand_table,
                       pos_table, ln_gamma, ln_beta, out, c, _NCHUNK)
    return out


# speedup vs baseline: 2.2782x; 1.0324x over previous
import functools

import jax
import jax.numpy as jnp
from jax import lax
from jax.experimental import pallas as pl
from jax.experimental.pallas import tpu as pltpu
from jax.experimental.pallas import tpu_sc as plsc

_NC = 2
_NS = 16
_NW = _NC * _NS
_LANES = 16
_NCHUNK = 8
_BB = 32


def _sc_gather(table_flat, ids_flat, base_row, n_rows, Hn, V, D):
    per_w = n_rows // _NW
    CH = 128
    n_iter = per_w // CH

    mesh = plsc.VectorSubcoreMesh(
        core_axis_name="c", subcore_axis_name="s",
        num_cores=_NC, num_subcores=_NS,
    )

    @functools.partial(
        pl.kernel,
        out_type=jax.ShapeDtypeStruct((n_rows, D), jnp.float32),
        mesh=mesh,
        scratch_types=[
            pltpu.VMEM((CH,), jnp.int32),
            pltpu.VMEM((CH, D), jnp.float32),
            pltpu.SemaphoreType.DMA,
        ],
    )
    def k(ids_hbm, table_hbm, out_hbm, idx_v, rows_v, sem):
        wid = lax.axis_index("s") * _NC + lax.axis_index("c")
        base = wid * per_w

        def body(i, carry):
            start = base + i * CH
            pltpu.sync_copy(ids_hbm.at[pl.ds(base_row + start, CH)], idx_v)

            def off_body(j, carry2):
                p = lax.iota(jnp.int32, 16) + (base_row + start + j * _LANES)
                h = lax.rem(p, Hn)
                sl = pl.ds(j * _LANES, _LANES)
                idx_v[sl] = idx_v[sl] + h * V
                return carry2

            lax.fori_loop(0, CH // _LANES, off_body, 0, unroll=True)
            pltpu.async_copy(table_hbm.at[idx_v], rows_v, sem).wait()
            pltpu.sync_copy(rows_v, out_hbm.at[pl.ds(start, CH)])
            return carry

        lax.fori_loop(0, n_iter, body, 0)

    return k(ids_flat, table_flat)


def _tc_post(gathered, strand_ids3, strand_table, pos_table, ln_gamma,
             ln_beta, out_prev, chunk, nchunk):
    Bc, Hn, D = gathered.shape
    B = Bc * nchunk
    grid = (Bc // _BB,)
    goff = chunk * (Bc // _BB)

    have_prev = out_prev is not None

    def body(g_ref, s_ref, st_ref, pt_ref, gm_ref, bt_ref, *rest):
        o_ref = rest[-1]
        x = g_ref[...].reshape(_BB, Hn, D)
        s = s_ref[0, 0, :].astype(jnp.float32)
        st = st_ref[...]
        semb = st[0][None, :] + s[:, None] * (st[1] - st[0])[None, :]
        x = x + pt_ref[...][None, :, :] + semb[:, None, :]
        mean = jnp.mean(x, axis=-1, keepdims=True)
        xc = x - mean
        var = jnp.mean(xc * xc, axis=-1, keepdims=True)
        y = xc * lax.rsqrt(var + 1e-5)
        o_ref[...] = y * gm_ref[...][None, None, :] + bt_ref[...][None, None, :]

    in_specs = [
        pl.BlockSpec((_BB * Hn, D), lambda i: (i, 0)),
        pl.BlockSpec((1, 1, _BB), lambda i: (i, 0, 0)),
        pl.BlockSpec((2, D), lambda i: (0, 0)),
        pl.BlockSpec((Hn, D), lambda i: (0, 0)),
        pl.BlockSpec((D,), lambda i: (0,)),
        pl.BlockSpec((D,), lambda i: (0,)),
    ]
    args = [gathered.reshape(Bc * Hn, D), strand_ids3, strand_table, pos_table,
            ln_gamma, ln_beta]
    if have_prev:
        in_specs.append(pl.BlockSpec(memory_space=pl.ANY))
        args.append(out_prev)
    return pl.pallas_call(
        body,
        grid=grid,
        in_specs=in_specs,
        out_specs=pl.BlockSpec((_BB, Hn, D), lambda i: (i + goff, 0, 0)),
        out_shape=jax.ShapeDtypeStruct((B, Hn, D), jnp.float32),
        input_output_aliases={6: 0} if have_prev else {},
    )(*args)


def kernel(cluster_ids, strand_ids, cluster_tables, strand_table, pos_table,
           ln_gamma, ln_beta):
    B, Hn = cluster_ids.shape
    _, V, D = cluster_tables.shape
    table_flat = cluster_tables.reshape(Hn * V, D)
    ids_flat = cluster_ids.reshape(B * Hn)
    Bc = B // _NCHUNK
    rows_c = Bc * Hn
    out = None
    for c in range(_NCHUNK):
        g = _sc_gather(table_flat, ids_flat, c * rows_c, rows_c, Hn, V, D)
        s3 = lax.dynamic_slice_in_dim(strand_ids, c * Bc, Bc).reshape(
            Bc // _BB, 1, _BB)
        out = _tc_post(g.reshape(Bc, Hn, D), s3, strand_table,
                       pos_table, ln_gamma, ln_beta, out, c, _NCHUNK)
    return out
